# baseline (speedup 1.0000x reference)
import functools

import jax
import jax.numpy as jnp
from jax import lax
from jax.experimental import pallas as pl
from jax.experimental.pallas import tpu as pltpu
from jax.experimental.pallas import tpu_sc as plsc

_NUM_CORES = 2
_NUM_SUBCORES = 16
_NUM_WORKERS = _NUM_CORES * _NUM_SUBCORES

_CHUNK = 128


@functools.partial(jax.jit, static_argnames=())
def kernel(input_ids, table):
    batch, hist = input_ids.shape
    vocab, dim = table.shape
    n = batch * hist
    assert n % _NUM_WORKERS == 0
    per_w = n // _NUM_WORKERS
    n_chunks = per_w // _CHUNK
    assert n_chunks * _CHUNK == per_w

    idx_flat = input_ids.reshape(n)

    mesh = plsc.VectorSubcoreMesh(
        core_axis_name="c", subcore_axis_name="s",
        num_cores=_NUM_CORES, num_subcores=_NUM_SUBCORES)

    @functools.partial(
        pl.kernel,
        out_type=jax.ShapeDtypeStruct((n, dim), jnp.float32),
        mesh=mesh,
        scratch_types=[
            pltpu.VMEM((per_w,), jnp.int32),
            pltpu.VMEM((_CHUNK, dim), jnp.float32),
            pltpu.SemaphoreType.DMA,
        ],
        compiler_params=pltpu.CompilerParams(use_tc_tiling_on_sc=False),
    )
    def gather_kernel(table_hbm, idx_hbm, out_hbm, idx_v, rows_v, sem):
        wid = lax.axis_index("s") * _NUM_CORES + lax.axis_index("c")
        base = wid * per_w
        pltpu.sync_copy(idx_hbm.at[pl.ds(base, per_w)], idx_v)

        def body(g, carry):
            off = g * _CHUNK
            pltpu.async_copy(
                table_hbm.at[idx_v.at[pl.ds(off, _CHUNK)]], rows_v, sem
            ).wait()
            pltpu.sync_copy(rows_v, out_hbm.at[pl.ds(base + off, _CHUNK)])
            return carry

        lax.fori_loop(0, n_chunks, body, 0)

    out = gather_kernel(table, idx_flat)
    return out.reshape(batch, hist, dim)


# speedup vs baseline: 1.1188x; 1.1188x over previous
import functools

import jax
import jax.numpy as jnp
from jax import lax
from jax.experimental import pallas as pl
from jax.experimental.pallas import tpu as pltpu
from jax.experimental.pallas import tpu_sc as plsc

_NUM_CORES = 2
_NUM_SUBCORES = 16
_NUM_WORKERS = _NUM_CORES * _NUM_SUBCORES

_CHUNK = 128
_NBUF = 4


def kernel(input_ids, table):
    batch, hist = input_ids.shape
    vocab, dim = table.shape
    n = batch * hist
    assert n % _NUM_WORKERS == 0
    per_w = n // _NUM_WORKERS
    n_chunks = per_w // _CHUNK
    assert n_chunks * _CHUNK == per_w and n_chunks % _NBUF == 0

    idx_flat = input_ids.reshape(n)

    mesh = plsc.VectorSubcoreMesh(
        core_axis_name="c", subcore_axis_name="s",
        num_cores=_NUM_CORES, num_subcores=_NUM_SUBCORES)

    @functools.partial(
        pl.kernel,
        out_type=jax.ShapeDtypeStruct((n, dim), jnp.float32),
        mesh=mesh,
        scratch_types=[
            pltpu.VMEM((per_w,), jnp.int32),
            [pltpu.VMEM((_CHUNK, dim), jnp.float32) for _ in range(_NBUF)],
            [pltpu.SemaphoreType.DMA for _ in range(_NBUF)],
        ],
        compiler_params=pltpu.CompilerParams(use_tc_tiling_on_sc=False),
    )
    def gather_kernel(table_hbm, idx_hbm, out_hbm, idx_v, bufs, sems):
        wid = lax.axis_index("s") * _NUM_CORES + lax.axis_index("c")
        base = wid * per_w
        pltpu.sync_copy(idx_hbm.at[pl.ds(base, per_w)], idx_v)

        def start_gather(g, b):
            pltpu.async_copy(
                table_hbm.at[idx_v.at[pl.ds(g * _CHUNK, _CHUNK)]],
                bufs[b], sems[b])

        def wait_gather(g, b):
            pltpu.make_async_copy(
                table_hbm.at[idx_v.at[pl.ds(g * _CHUNK, _CHUNK)]],
                bufs[b], sems[b]).wait()

        for b in range(_NBUF):
            start_gather(b, b)

        def round_body(r, carry):
            g0 = r * _NBUF
            for b in range(_NBUF):
                g = g0 + b
                wait_gather(g, b)
                pltpu.sync_copy(bufs[b], out_hbm.at[pl.ds(base + g * _CHUNK, _CHUNK)])
                start_gather(g + _NBUF, b)
            return carry

        lax.fori_loop(0, (n_chunks - _NBUF) // _NBUF, round_body, 0,
                      unroll=False)

        g0 = n_chunks - _NBUF
        for b in range(_NBUF):
            g = g0 + b
            wait_gather(g, b)
            pltpu.sync_copy(bufs[b], out_hbm.at[pl.ds(base + g * _CHUNK, _CHUNK)])

    out = gather_kernel(table, idx_flat)
    return out.reshape(batch, hist, dim)
